# baseline (speedup 1.0000x reference)
import functools
import math

import jax
import jax.numpy as jnp
from jax import lax
from jax.experimental import pallas as pl
from jax.experimental.pallas import tpu as pltpu
from jax.experimental.pallas import tpu_sc as plsc

D_MODEL = 768
SEQ = 4096
BATCH = 4
TOKENS = BATCH * SEQ
SCALE = math.sqrt(float(D_MODEL))

_INFO = plsc.get_sparse_core_info()
NUM_WORKERS = _INFO.num_cores * _INFO.num_subcores
TPW = TOKENS // NUM_WORKERS
CB = 64
NCHUNK = TPW // CB
VPR = D_MODEL // 16


def _pe_table(seq_len, d_model):
    pos = jnp.arange(seq_len, dtype=jnp.float32)[:, None]
    div = jnp.exp(
        jnp.arange(0, d_model, 2, dtype=jnp.float32)
        * (-math.log(10000.0) / d_model)
    )
    pe = jnp.zeros((seq_len, d_model), dtype=jnp.float32)
    pe = pe.at[:, 0::2].set(jnp.sin(pos * div))
    pe = pe.at[:, 1::2].set(jnp.cos(pos * div))
    return pe


def _sc_body(idx_hbm, table_hbm, pe_hbm, out_hbm, idx_v, rows_v, pe_v, sem):
    wid = lax.axis_index("s") * _INFO.num_cores + lax.axis_index("c")
    base = wid * TPW
    pos0 = (wid % (SEQ // TPW)) * TPW
    pltpu.sync_copy(idx_hbm.at[pl.ds(base, TPW)], idx_v)
    for c in range(NCHUNK):
        gather = pltpu.async_copy(
            table_hbm.at[idx_v.at[pl.ds(c * CB, CB)]], rows_v, sem)
        pltpu.sync_copy(pe_hbm.at[pl.ds(pos0 + c * CB, CB)], pe_v)
        gather.wait()

        def fma_row(i, carry):
            for j in range(VPR):
                sl = pl.ds(j * 16, 16)
                rows_v[i, sl] = rows_v[i, sl] * SCALE + pe_v[i, sl]
            return carry

        lax.fori_loop(0, CB, fma_row, 0)
        pltpu.sync_copy(rows_v, out_hbm.at[pl.ds(base + c * CB, CB)])


def kernel(x, table):
    idx = x.reshape(-1).astype(jnp.int32)
    pe = _pe_table(SEQ, D_MODEL)
    mesh = plsc.VectorSubcoreMesh(core_axis_name="c", subcore_axis_name="s")
    run = functools.partial(
        pl.kernel,
        out_type=jax.ShapeDtypeStruct((TOKENS, D_MODEL), jnp.float32),
        mesh=mesh,
        scratch_types=[
            pltpu.VMEM((TPW,), jnp.int32),
            pltpu.VMEM((CB, D_MODEL), jnp.float32),
            pltpu.VMEM((CB, D_MODEL), jnp.float32),
            pltpu.SemaphoreType.DMA,
        ],
    )(_sc_body)
    out = run(idx, table, pe)
    return out.reshape(BATCH, SEQ, D_MODEL)


# speedup vs baseline: 1.1834x; 1.1834x over previous
import functools
import math

import jax
import jax.numpy as jnp
from jax import lax
from jax.experimental import pallas as pl
from jax.experimental.pallas import tpu as pltpu
from jax.experimental.pallas import tpu_sc as plsc

D_MODEL = 768
SEQ = 4096
BATCH = 4
TOKENS = BATCH * SEQ
SCALE = math.sqrt(float(D_MODEL))

_INFO = plsc.get_sparse_core_info()
NUM_WORKERS = _INFO.num_cores * _INFO.num_subcores
TPW = TOKENS // NUM_WORKERS
CB = 32
NCHUNK = TPW // CB
VPR = D_MODEL // 16
NROW = 3
NPE = 2


def _pe_table(seq_len, d_model):
    pos = jnp.arange(seq_len, dtype=jnp.float32)[:, None]
    div = jnp.exp(
        jnp.arange(0, d_model, 2, dtype=jnp.float32)
        * (-math.log(10000.0) / d_model)
    )
    pe = jnp.zeros((seq_len, d_model), dtype=jnp.float32)
    pe = pe.at[:, 0::2].set(jnp.sin(pos * div))
    pe = pe.at[:, 1::2].set(jnp.cos(pos * div))
    return pe


def _sc_body(idx_hbm, table_hbm, pe_hbm, out_hbm, idx_v, *scratch):
    rows = scratch[0:NROW]
    pes = scratch[NROW:NROW + NPE]
    gsem = scratch[NROW + NPE:2 * NROW + NPE]
    psem = scratch[2 * NROW + NPE:2 * NROW + 2 * NPE]
    ssem = scratch[2 * NROW + 2 * NPE:3 * NROW + 2 * NPE]

    wid = lax.axis_index("s") * _INFO.num_cores + lax.axis_index("c")
    base = wid * TPW
    pos0 = (wid % (SEQ // TPW)) * TPW
    pltpu.sync_copy(idx_hbm.at[pl.ds(base, TPW)], idx_v)

    def issue(c):
        b = c % NROW
        g = pltpu.async_copy(
            table_hbm.at[idx_v.at[pl.ds(c * CB, CB)]], rows[b], gsem[b])
        p = pltpu.async_copy(
            pe_hbm.at[pl.ds(pos0 + c * CB, CB)], pes[c % NPE], psem[c % NPE])
        return g, p

    inflight = {0: issue(0)}
    stores = {}
    for c in range(NCHUNK):
        if c + 1 < NCHUNK:
            b1 = (c + 1) % NROW
            if c + 1 >= NROW:
                stores.pop(c + 1 - NROW).wait()
            inflight[c + 1] = issue(c + 1)
        g, p = inflight.pop(c)
        g.wait()
        p.wait()
        rbuf, pbuf = rows[c % NROW], pes[c % NPE]

        def fma_row(i, carry):
            for j in range(VPR):
                sl = pl.ds(j * 16, 16)
                rbuf[i, sl] = rbuf[i, sl] * SCALE + pbuf[i, sl]
            return carry

        lax.fori_loop(0, CB, fma_row, 0)
        stores[c] = pltpu.async_copy(
            rows[c % NROW], out_hbm.at[pl.ds(base + c * CB, CB)],
            ssem[c % NROW])
    for c in sorted(stores):
        stores[c].wait()


def kernel(x, table):
    idx = x.reshape(-1).astype(jnp.int32)
    pe = _pe_table(SEQ, D_MODEL)
    mesh = plsc.VectorSubcoreMesh(core_axis_name="c", subcore_axis_name="s")
    scratch = (
        [pltpu.VMEM((TPW,), jnp.int32)]
        + [pltpu.VMEM((CB, D_MODEL), jnp.float32) for _ in range(NROW)]
        + [pltpu.VMEM((CB, D_MODEL), jnp.float32) for _ in range(NPE)]
        + [pltpu.SemaphoreType.DMA for _ in range(2 * NROW + 2 * NPE)]
    )
    run = functools.partial(
        pl.kernel,
        out_type=jax.ShapeDtypeStruct((TOKENS, D_MODEL), jnp.float32),
        mesh=mesh,
        scratch_types=scratch,
    )(_sc_body)
    out = run(idx, table, pe)
    return out.reshape(BATCH, SEQ, D_MODEL)


# speedup vs baseline: 2.0381x; 1.7222x over previous
import functools
import math

import numpy as np

import jax
import jax.numpy as jnp
from jax import lax
from jax.experimental import pallas as pl
from jax.experimental.pallas import tpu as pltpu
from jax.experimental.pallas import tpu_sc as plsc

D_MODEL = 768
SEQ = 4096
BATCH = 4
TOKENS = BATCH * SEQ
SCALE = math.sqrt(float(D_MODEL))

_INFO = plsc.get_sparse_core_info()
NUM_WORKERS = _INFO.num_cores * _INFO.num_subcores
TPW = TOKENS // NUM_WORKERS
CB = 32
NCHUNK = TPW // CB
VPR = D_MODEL // 16
NROW = 3
NPE = 2


def _pe_table(seq_len, d_model):
    pos = np.arange(seq_len, dtype=np.float32)[:, None]
    div = np.exp(
        np.arange(0, d_model, 2, dtype=np.float32)
        * (-math.log(10000.0) / d_model)
    )
    pe = np.zeros((seq_len, d_model), dtype=np.float32)
    pe[:, 0::2] = np.sin(pos * div)
    pe[:, 1::2] = np.cos(pos * div)
    return jnp.asarray(pe)


def _sc_body(idx_hbm, table_hbm, pe_hbm, out_hbm, idx_v, *scratch):
    rows = scratch[0:NROW]
    pes = scratch[NROW:NROW + NPE]
    gsem = scratch[NROW + NPE:2 * NROW + NPE]
    psem = scratch[2 * NROW + NPE:2 * NROW + 2 * NPE]
    ssem = scratch[2 * NROW + 2 * NPE:3 * NROW + 2 * NPE]

    wid = lax.axis_index("s") * _INFO.num_cores + lax.axis_index("c")
    base = wid * TPW
    pos0 = (wid % (SEQ // TPW)) * TPW
    pltpu.sync_copy(idx_hbm.at[pl.ds(base, TPW)], idx_v)

    def issue(c):
        b = c % NROW
        g = pltpu.async_copy(
            table_hbm.at[idx_v.at[pl.ds(c * CB, CB)]], rows[b], gsem[b])
        p = pltpu.async_copy(
            pe_hbm.at[pl.ds(pos0 + c * CB, CB)], pes[c % NPE], psem[c % NPE])
        return g, p

    inflight = {0: issue(0)}
    stores = {}
    for c in range(NCHUNK):
        if c + 1 < NCHUNK:
            b1 = (c + 1) % NROW
            if c + 1 >= NROW:
                stores.pop(c + 1 - NROW).wait()
            inflight[c + 1] = issue(c + 1)
        g, p = inflight.pop(c)
        g.wait()
        p.wait()
        rbuf, pbuf = rows[c % NROW], pes[c % NPE]

        def fma_row(i, carry):
            for j in range(VPR):
                sl = pl.ds(j * 16, 16)
                rbuf[i, sl] = rbuf[i, sl] * SCALE + pbuf[i, sl]
            return carry

        lax.fori_loop(0, CB, fma_row, 0)
        stores[c] = pltpu.async_copy(
            rows[c % NROW], out_hbm.at[pl.ds(base + c * CB, CB)],
            ssem[c % NROW])
    for c in sorted(stores):
        stores[c].wait()


def kernel(x, table):
    idx = x.reshape(-1).astype(jnp.int32)
    pe = _pe_table(SEQ, D_MODEL)
    mesh = plsc.VectorSubcoreMesh(core_axis_name="c", subcore_axis_name="s")
    scratch = (
        [pltpu.VMEM((TPW,), jnp.int32)]
        + [pltpu.VMEM((CB, D_MODEL), jnp.float32) for _ in range(NROW)]
        + [pltpu.VMEM((CB, D_MODEL), jnp.float32) for _ in range(NPE)]
        + [pltpu.SemaphoreType.DMA for _ in range(2 * NROW + 2 * NPE)]
    )
    run = functools.partial(
        pl.kernel,
        out_type=jax.ShapeDtypeStruct((TOKENS, D_MODEL), jnp.float32),
        mesh=mesh,
        scratch_types=scratch,
    )(_sc_body)
    out = run(idx, table, pe)
    return out.reshape(BATCH, SEQ, D_MODEL)
